# baseline (speedup 1.0000x reference)
import functools

import jax
import jax.numpy as jnp
from jax import lax
from jax.experimental import pallas as pl
from jax.experimental.pallas import tpu as pltpu
from jax.experimental.pallas import tpu_sc as plsc

_XFER = 128
_NUM_CORES = 2
_NUM_SUBCORES = 16
_NW = _NUM_CORES * _NUM_SUBCORES
_K = 4


@functools.lru_cache(maxsize=None)
def _build_gather(n_xfer_rows: int, d: int):
    xpw = n_xfer_rows // _NW
    chunks = []
    s = 0
    while s < xpw:
        c = min(_K, xpw - s)
        chunks.append((s, c))
        s += c
    mesh = plsc.VectorSubcoreMesh(core_axis_name="c", subcore_axis_name="s")

    @functools.partial(
        pl.kernel,
        mesh=mesh,
        out_type=jax.ShapeDtypeStruct((n_xfer_rows * _XFER, d), jnp.float32),
        scratch_types=[
            pltpu.VMEM((xpw, _XFER), jnp.int32),
            pltpu.VMEM((2, _K * _XFER, d), jnp.float32),
            pltpu.SemaphoreType.DMA,
            pltpu.SemaphoreType.DMA,
            pltpu.SemaphoreType.DMA,
            pltpu.SemaphoreType.DMA,
        ],
        compiler_params=pltpu.CompilerParams(use_tc_tiling_on_sc=False),
    )
    def gather(idx_hbm, table_hbm, out_hbm, idx_v, rows_v,
               gsem0, gsem1, wsem0, wsem1):
        gsems = (gsem0, gsem1)
        wsems = (wsem0, wsem1)
        wid = lax.axis_index("s") * _NUM_CORES + lax.axis_index("c")
        row0 = wid * xpw * _XFER
        pltpu.sync_copy(idx_hbm.at[wid], idx_v)

        def fire(ci):
            s0, c = chunks[ci]
            buf = ci % 2
            return [
                pltpu.async_copy(
                    table_hbm.at[idx_v.at[s0 + j]],
                    rows_v.at[buf, pl.ds(j * _XFER, _XFER)],
                    gsems[buf],
                )
                for j in range(c)
            ]

        wpend = [None, None]
        pending = fire(0)
        for ci in range(len(chunks)):
            buf = ci % 2
            if ci + 1 < len(chunks):
                if wpend[1 - buf] is not None:
                    wpend[1 - buf].wait()
                    wpend[1 - buf] = None
                nxt = fire(ci + 1)
            else:
                nxt = []
            for cp in pending:
                cp.wait()
            s0, c = chunks[ci]
            wpend[buf] = pltpu.async_copy(
                rows_v.at[buf, pl.ds(0, c * _XFER)],
                out_hbm.at[pl.ds(row0 + s0 * _XFER, c * _XFER)],
                wsems[buf],
            )
            pending = nxt
        for w in wpend:
            if w is not None:
                w.wait()

    return gather


def kernel(inputs, embeddings):
    b, s = inputs.shape
    n = b * s
    d = embeddings.shape[1]
    idx3d = inputs.reshape(_NW, n // (_NW * _XFER), _XFER).astype(jnp.int32)
    out = _build_gather(n // _XFER, d)(idx3d, embeddings)
    return out.reshape(b, s, d)
